# baseline (speedup 1.0000x reference)
import jax
import jax.numpy as jnp
from jax import lax
from jax.experimental import pallas as pl
from jax.experimental.pallas import tpu as pltpu
from jax.experimental.pallas import tpu_sc as plsc

_N = 10000
_E = 320000
_D = 128
_G = 64
_ER = 5120
_RPT = _ER // 32
_NA = _N + 16
_DEGN = 10240
_NBUF = 4
_NIDX = 2 * _NBUF


def _sc_body(feat, ed3, p_out, dg_out,
             eidx, rows, zbuf, zflat, ones,
             acc_sh, deg_sh, isem, gsem, ssem, dsem):
    c = lax.axis_index("c")
    s = lax.axis_index("s")
    w = c * 16 + s
    tb = w * _RPT

    zf = jnp.zeros((16,), jnp.float32)

    scope = jax.named_scope

    _s0 = scope("zero_phase"); _s0.__enter__()

    def _zb(i, _):
        for k in range(_D // 16):
            zbuf[i, pl.ds(k * 16, 16)] = zf
        return 0
    lax.fori_loop(0, 16, _zb, 0)

    def _zfl(i, _):
        zflat[pl.ds(i * 16, 16)] = zf
        return 0
    lax.fori_loop(0, 1280 // 16, _zfl, 0)

    def _zacc(q, _):
        ci = q * 16 + s
        @pl.when(ci < _NA // 16)
        def _():
            off = pl.multiple_of(ci * 16, 16)
            pltpu.sync_copy(zbuf, acc_sh.at[pl.ds(off, 16)])
        return 0
    lax.fori_loop(0, (_NA // 16 + 15) // 16, _zacc, 0)

    @pl.when(s < 8)
    def _():
        pltpu.sync_copy(zflat, deg_sh.at[pl.ds(s * 1280, 1280)])

    for k in range(_G // 16):
        ones[pl.ds(k * 16, 16)] = jnp.full((16,), 1.0, jnp.float32)

    _s0.__exit__(None, None, None)
    with scope("barrier0"):
        plsc.subcore_barrier()

    _s1 = scope("edge_phase"); _s1.__enter__()
    for b0 in range(_NIDX):
        pltpu.async_copy(ed3.at[tb + b0], eidx.at[b0], isem.at[b0])
    for j in range(_NBUF):
        pltpu.make_async_copy(ed3.at[tb + j], eidx.at[j], isem.at[j]).wait()
        pltpu.async_copy(feat.at[eidx.at[j, 0]], rows.at[j], gsem.at[j])

    rpt_w = jnp.where(w == 31, _E // _G - 31 * _RPT, _RPT)

    def _outer(q, _):
        base = q * _NBUF
        descs = []
        for j in range(_NBUF):
            b = base + j
            islot = lax.rem(b, _NIDX)
            pltpu.make_async_copy(feat.at[eidx.at[islot, 0]],
                                  rows.at[j], gsem.at[j]).wait()
            s1 = pltpu.async_copy(rows.at[j], acc_sh.at[eidx.at[islot, 1]],
                                  ssem.at[j], add=True)
            s2 = pltpu.async_copy(ones, deg_sh.at[eidx.at[islot, 1]],
                                  dsem.at[j], add=True)
            descs.append((s1, s2))
        for j in range(_NBUF):
            b = base + j
            islot = lax.rem(b, _NIDX)
            s1, s2 = descs[j]
            s1.wait()
            s2.wait()
            @pl.when(b + _NIDX < rpt_w)
            def _():
                pltpu.async_copy(ed3.at[tb + b + _NIDX], eidx.at[islot],
                                 isem.at[islot])
        for j in range(_NBUF):
            b = base + j + _NBUF
            islot = lax.rem(b, _NIDX)
            @pl.when(b < rpt_w)
            def _():
                pltpu.make_async_copy(ed3.at[tb + b], eidx.at[islot],
                                      isem.at[islot]).wait()
                pltpu.async_copy(feat.at[eidx.at[islot, 0]],
                                 rows.at[j], gsem.at[j])
        return 0
    lax.fori_loop(0, rpt_w // _NBUF, _outer, 0)

    _s1.__exit__(None, None, None)
    with scope("barrier1"):
        plsc.subcore_barrier()

    _s2 = scope("dump_phase"); _s2.__enter__()
    @pl.when(s < 15)
    def _():
        off = pl.multiple_of(s * 624, 8)
        pltpu.sync_copy(acc_sh.at[pl.ds(off, 624)],
                        p_out.at[pl.ds(c * _N + off, 624)])

    @pl.when(s == 15)
    def _():
        pltpu.sync_copy(acc_sh.at[pl.ds(9360, 640)],
                        p_out.at[pl.ds(c * _N + 9360, 640)])

    @pl.when(s < 7)
    def _():
        pltpu.sync_copy(deg_sh.at[pl.ds(s * 1280, 1280)], zflat)
        pltpu.sync_copy(zflat, dg_out.at[pl.ds(c * _N + s * 1280, 1280)])

    @pl.when(s == 7)
    def _():
        pltpu.sync_copy(deg_sh.at[pl.ds(8960, 1040)],
                        zflat.at[pl.ds(0, 1040)])
        pltpu.sync_copy(zflat.at[pl.ds(0, 1040)],
                        dg_out.at[pl.ds(c * _N + 8960, 1040)])

    _s2.__exit__(None, None, None)


def _sc_part(feat, ed3):
    mesh = plsc.VectorSubcoreMesh(core_axis_name="c", subcore_axis_name="s")
    fn = pl.kernel(
        _sc_body,
        out_type=(jax.ShapeDtypeStruct((2 * _N, _D), jnp.float32),
                  jax.ShapeDtypeStruct((2 * _N,), jnp.float32)),
        mesh=mesh,
        scratch_types=[
            pltpu.VMEM((_NIDX, 2, _G), jnp.int32),
            pltpu.VMEM((_NBUF, _G, _D), jnp.float32),
            pltpu.VMEM((16, _D), jnp.float32),
            pltpu.VMEM((1280,), jnp.float32),
            pltpu.VMEM((_G,), jnp.float32),
            pltpu.VMEM_SHARED((_NA, _D), jnp.float32),
            pltpu.VMEM_SHARED((_DEGN,), jnp.float32),
            pltpu.SemaphoreType.DMA((_NIDX,)),
            pltpu.SemaphoreType.DMA((_NBUF,)),
            pltpu.SemaphoreType.DMA((_NBUF,)),
            pltpu.SemaphoreType.DMA((_NBUF,)),
        ],
    )
    return fn(feat, ed3)


def _tc_combine_body(p0_ref, p1_ref, d0_ref, d1_ref, f_ref, o_ref):
    d = d0_ref[...] + d1_ref[...]
    o_ref[...] = p0_ref[...] + p1_ref[...] + d * f_ref[...]


def _tc_combine(p2, dg2, feat):
    blk = 400
    nb = _N // blk
    dg2 = dg2.reshape(2 * _N, 1)
    return pl.pallas_call(
        _tc_combine_body,
        out_shape=jax.ShapeDtypeStruct((_N, _D), jnp.float32),
        grid=(nb,),
        in_specs=[
            pl.BlockSpec((blk, _D), lambda i: (i, 0)),
            pl.BlockSpec((blk, _D), lambda i: (i + nb, 0)),
            pl.BlockSpec((blk, 1), lambda i: (i, 0)),
            pl.BlockSpec((blk, 1), lambda i: (i + nb, 0)),
            pl.BlockSpec((blk, _D), lambda i: (i, 0)),
        ],
        out_specs=pl.BlockSpec((blk, _D), lambda i: (i, 0)),
    )(p2, p2, dg2, dg2, feat)


def kernel(node_feat, edge_index):
    npad = _ER * _G - _E
    src2 = jnp.pad(edge_index[0], (0, npad)).reshape(_ER, _G)
    dst2 = jnp.pad(edge_index[1], (0, npad),
                   constant_values=_N).reshape(_ER, _G)
    ed3 = jnp.stack([src2, dst2], axis=1)
    p2, dg2 = _sc_part(node_feat, ed3)
    return _tc_combine(p2, dg2, node_feat)


# speedup vs baseline: 1.0239x; 1.0239x over previous
import jax
import jax.numpy as jnp
from jax import lax
from jax.experimental import pallas as pl
from jax.experimental.pallas import tpu as pltpu
from jax.experimental.pallas import tpu_sc as plsc

_N = 10000
_E = 320000
_D = 128
_G = 128
_ER = 2560
_RPT = _ER // 32
_NA = _N + 16
_DEGN = 10240
_NBUF = 2
_NIDX = 2 * _NBUF


def _sc_body(feat, ed3, p_out, dg_out,
             eidx, rows, zbuf, zflat, ones,
             acc_sh, deg_sh, isem, gsem, ssem, dsem):
    c = lax.axis_index("c")
    s = lax.axis_index("s")
    w = c * 16 + s
    tb = w * _RPT

    zf = jnp.zeros((16,), jnp.float32)

    scope = jax.named_scope

    _s0 = scope("zero_phase"); _s0.__enter__()

    def _zb(i, _):
        for k in range(_D // 16):
            zbuf[i, pl.ds(k * 16, 16)] = zf
        return 0
    lax.fori_loop(0, 16, _zb, 0)

    def _zfl(i, _):
        zflat[pl.ds(i * 16, 16)] = zf
        return 0
    lax.fori_loop(0, 1280 // 16, _zfl, 0)

    def _zacc(q, _):
        ci = q * 16 + s
        @pl.when(ci < _NA // 16)
        def _():
            off = pl.multiple_of(ci * 16, 16)
            pltpu.sync_copy(zbuf, acc_sh.at[pl.ds(off, 16)])
        return 0
    lax.fori_loop(0, (_NA // 16 + 15) // 16, _zacc, 0)

    @pl.when(s < 8)
    def _():
        pltpu.sync_copy(zflat, deg_sh.at[pl.ds(s * 1280, 1280)])

    for k in range(_G // 16):
        ones[pl.ds(k * 16, 16)] = jnp.full((16,), 1.0, jnp.float32)

    _s0.__exit__(None, None, None)
    with scope("barrier0"):
        plsc.subcore_barrier()

    _s1 = scope("edge_phase"); _s1.__enter__()
    for b0 in range(_NIDX):
        pltpu.async_copy(ed3.at[tb + b0], eidx.at[b0], isem.at[b0])
    for j in range(_NBUF):
        pltpu.make_async_copy(ed3.at[tb + j], eidx.at[j], isem.at[j]).wait()
        pltpu.async_copy(feat.at[eidx.at[j, 0]], rows.at[j], gsem.at[j])

    rpt_w = jnp.where(w == 31, (_ER - 60) - 31 * _RPT, _RPT)

    def _outer(q, _):
        base = q * _NBUF
        descs = []
        for j in range(_NBUF):
            b = base + j
            islot = lax.rem(b, _NIDX)
            pltpu.make_async_copy(feat.at[eidx.at[islot, 0]],
                                  rows.at[j], gsem.at[j]).wait()
            s1 = pltpu.async_copy(rows.at[j], acc_sh.at[eidx.at[islot, 1]],
                                  ssem.at[j], add=True)
            s2 = pltpu.async_copy(ones, deg_sh.at[eidx.at[islot, 1]],
                                  dsem.at[j], add=True)
            descs.append((s1, s2))
        for j in range(_NBUF):
            b = base + j
            islot = lax.rem(b, _NIDX)
            s1, s2 = descs[j]
            s1.wait()
            s2.wait()
            @pl.when(b + _NIDX < rpt_w)
            def _():
                pltpu.async_copy(ed3.at[tb + b + _NIDX], eidx.at[islot],
                                 isem.at[islot])
        for j in range(_NBUF):
            b = base + j + _NBUF
            islot = lax.rem(b, _NIDX)
            @pl.when(b < rpt_w)
            def _():
                pltpu.make_async_copy(ed3.at[tb + b], eidx.at[islot],
                                      isem.at[islot]).wait()
                pltpu.async_copy(feat.at[eidx.at[islot, 0]],
                                 rows.at[j], gsem.at[j])
        return 0
    lax.fori_loop(0, rpt_w // _NBUF, _outer, 0)

    _s1.__exit__(None, None, None)
    with scope("barrier1"):
        plsc.subcore_barrier()

    _s2 = scope("dump_phase"); _s2.__enter__()
    @pl.when(s < 15)
    def _():
        off = pl.multiple_of(s * 624, 8)
        pltpu.sync_copy(acc_sh.at[pl.ds(off, 624)],
                        p_out.at[pl.ds(c * _N + off, 624)])

    @pl.when(s == 15)
    def _():
        pltpu.sync_copy(acc_sh.at[pl.ds(9360, 640)],
                        p_out.at[pl.ds(c * _N + 9360, 640)])

    @pl.when(s < 7)
    def _():
        pltpu.sync_copy(deg_sh.at[pl.ds(s * 1280, 1280)], zflat)
        pltpu.sync_copy(zflat, dg_out.at[pl.ds(c * _N + s * 1280, 1280)])

    @pl.when(s == 7)
    def _():
        pltpu.sync_copy(deg_sh.at[pl.ds(8960, 1040)],
                        zflat.at[pl.ds(0, 1040)])
        pltpu.sync_copy(zflat.at[pl.ds(0, 1040)],
                        dg_out.at[pl.ds(c * _N + 8960, 1040)])

    _s2.__exit__(None, None, None)


def _sc_part(feat, ed3):
    mesh = plsc.VectorSubcoreMesh(core_axis_name="c", subcore_axis_name="s")
    fn = pl.kernel(
        _sc_body,
        out_type=(jax.ShapeDtypeStruct((2 * _N, _D), jnp.float32),
                  jax.ShapeDtypeStruct((2 * _N,), jnp.float32)),
        mesh=mesh,
        scratch_types=[
            pltpu.VMEM((_NIDX, 2, _G), jnp.int32),
            pltpu.VMEM((_NBUF, _G, _D), jnp.float32),
            pltpu.VMEM((16, _D), jnp.float32),
            pltpu.VMEM((1280,), jnp.float32),
            pltpu.VMEM((_G,), jnp.float32),
            pltpu.VMEM_SHARED((_NA, _D), jnp.float32),
            pltpu.VMEM_SHARED((_DEGN,), jnp.float32),
            pltpu.SemaphoreType.DMA((_NIDX,)),
            pltpu.SemaphoreType.DMA((_NBUF,)),
            pltpu.SemaphoreType.DMA((_NBUF,)),
            pltpu.SemaphoreType.DMA((_NBUF,)),
        ],
    )
    return fn(feat, ed3)


def _tc_combine_body(p0_ref, p1_ref, d0_ref, d1_ref, f_ref, o_ref):
    d = d0_ref[...] + d1_ref[...]
    o_ref[...] = p0_ref[...] + p1_ref[...] + d * f_ref[...]


def _tc_combine(p2, dg2, feat):
    blk = 400
    nb = _N // blk
    dg2 = dg2.reshape(2 * _N, 1)
    return pl.pallas_call(
        _tc_combine_body,
        out_shape=jax.ShapeDtypeStruct((_N, _D), jnp.float32),
        grid=(nb,),
        in_specs=[
            pl.BlockSpec((blk, _D), lambda i: (i, 0)),
            pl.BlockSpec((blk, _D), lambda i: (i + nb, 0)),
            pl.BlockSpec((blk, 1), lambda i: (i, 0)),
            pl.BlockSpec((blk, 1), lambda i: (i + nb, 0)),
            pl.BlockSpec((blk, _D), lambda i: (i, 0)),
        ],
        out_specs=pl.BlockSpec((blk, _D), lambda i: (i, 0)),
    )(p2, p2, dg2, dg2, feat)


def kernel(node_feat, edge_index):
    npad = _ER * _G - _E
    src2 = jnp.pad(edge_index[0], (0, npad)).reshape(_ER, _G)
    dst2 = jnp.pad(edge_index[1], (0, npad),
                   constant_values=_N).reshape(_ER, _G)
    ed3 = jnp.stack([src2, dst2], axis=1)
    p2, dg2 = _sc_part(node_feat, ed3)
    return _tc_combine(p2, dg2, node_feat)


# speedup vs baseline: 1.0868x; 1.0614x over previous
import jax
import jax.numpy as jnp
from jax import lax
from jax.experimental import pallas as pl
from jax.experimental.pallas import tpu as pltpu
from jax.experimental.pallas import tpu_sc as plsc

_N = 10000
_E = 320000
_D = 128
_G = 128
_ER = _E // _G
_RPT = 78
_NA = _N + 16
_DEGN = 10240
_NBUF = 2
_NIDX = 2 * _NBUF


def _sc_body(feat, src3, dst3, p_out, dg_out,
             sidx, didx, rows, zbuf, zflat, ones,
             acc_sh, deg_sh, isem, gsem, ssem, dsem):
    c = lax.axis_index("c")
    s = lax.axis_index("s")
    w = c * 16 + s
    tb = w * _RPT

    zf = jnp.zeros((16,), jnp.float32)

    scope = jax.named_scope

    _s0 = scope("zero_phase"); _s0.__enter__()

    def _zb(i, _):
        for k in range(_D // 16):
            zbuf[i, pl.ds(k * 16, 16)] = zf
        return 0
    lax.fori_loop(0, 16, _zb, 0)

    def _zfl(i, _):
        zflat[pl.ds(i * 16, 16)] = zf
        return 0
    lax.fori_loop(0, 1280 // 16, _zfl, 0)

    def _zacc(q, _):
        ci = q * 16 + s
        @pl.when(ci < _NA // 16)
        def _():
            off = pl.multiple_of(ci * 16, 16)
            pltpu.sync_copy(zbuf, acc_sh.at[pl.ds(off, 16)])
        return 0
    lax.fori_loop(0, (_NA // 16 + 15) // 16, _zacc, 0)

    @pl.when(s < 8)
    def _():
        pltpu.sync_copy(zflat, deg_sh.at[pl.ds(s * 1280, 1280)])

    for k in range(_G // 16):
        ones[pl.ds(k * 16, 16)] = jnp.full((16,), 1.0, jnp.float32)

    _s0.__exit__(None, None, None)
    with scope("barrier0"):
        plsc.subcore_barrier()

    _s1 = scope("edge_phase"); _s1.__enter__()
    for b0 in range(_NIDX):
        pltpu.async_copy(src3.at[tb + b0], sidx.at[b0], isem.at[b0])
        pltpu.async_copy(dst3.at[tb + b0], didx.at[b0], isem.at[b0])
    for j in range(_NBUF):
        pltpu.make_async_copy(src3.at[tb + j], sidx.at[j], isem.at[j]).wait()
        pltpu.make_async_copy(dst3.at[tb + j], didx.at[j], isem.at[j]).wait()
        pltpu.async_copy(feat.at[sidx.at[j, 0]], rows.at[j], gsem.at[j])

    rpt_w = jnp.where(w == 31, _ER - 31 * _RPT, _RPT)

    def _outer(q, _):
        base = q * _NBUF
        descs = []
        for j in range(_NBUF):
            b = base + j
            islot = lax.rem(b, _NIDX)
            pltpu.make_async_copy(feat.at[sidx.at[islot, 0]],
                                  rows.at[j], gsem.at[j]).wait()
            s1 = pltpu.async_copy(rows.at[j], acc_sh.at[didx.at[islot, 0]],
                                  ssem.at[j], add=True)
            s2 = pltpu.async_copy(ones, deg_sh.at[didx.at[islot, 0]],
                                  dsem.at[j], add=True)
            descs.append((s1, s2))
        for j in range(_NBUF):
            b = base + j
            islot = lax.rem(b, _NIDX)
            s1, s2 = descs[j]
            s1.wait()
            s2.wait()
            @pl.when(b + _NIDX < rpt_w)
            def _():
                pltpu.async_copy(src3.at[tb + b + _NIDX], sidx.at[islot],
                                 isem.at[islot])
                pltpu.async_copy(dst3.at[tb + b + _NIDX], didx.at[islot],
                                 isem.at[islot])
        for j in range(_NBUF):
            b = base + j + _NBUF
            islot = lax.rem(b, _NIDX)
            @pl.when(b < rpt_w)
            def _():
                pltpu.make_async_copy(src3.at[tb + b], sidx.at[islot],
                                      isem.at[islot]).wait()
                pltpu.make_async_copy(dst3.at[tb + b], didx.at[islot],
                                      isem.at[islot]).wait()
                pltpu.async_copy(feat.at[sidx.at[islot, 0]],
                                 rows.at[j], gsem.at[j])
        return 0
    lax.fori_loop(0, rpt_w // _NBUF, _outer, 0)

    _s1.__exit__(None, None, None)
    with scope("barrier1"):
        plsc.subcore_barrier()

    _s2 = scope("dump_phase"); _s2.__enter__()
    @pl.when(s < 15)
    def _():
        off = pl.multiple_of(s * 624, 8)
        pltpu.sync_copy(acc_sh.at[pl.ds(off, 624)],
                        p_out.at[pl.ds(c * _N + off, 624)])

    @pl.when(s == 15)
    def _():
        pltpu.sync_copy(acc_sh.at[pl.ds(9360, 640)],
                        p_out.at[pl.ds(c * _N + 9360, 640)])

    @pl.when(s < 7)
    def _():
        pltpu.sync_copy(deg_sh.at[pl.ds(s * 1280, 1280)], zflat)
        pltpu.sync_copy(zflat, dg_out.at[pl.ds(c * _N + s * 1280, 1280)])

    @pl.when(s == 7)
    def _():
        pltpu.sync_copy(deg_sh.at[pl.ds(8960, 1040)],
                        zflat.at[pl.ds(0, 1040)])
        pltpu.sync_copy(zflat.at[pl.ds(0, 1040)],
                        dg_out.at[pl.ds(c * _N + 8960, 1040)])

    _s2.__exit__(None, None, None)


def _sc_part(feat, src3, dst3):
    mesh = plsc.VectorSubcoreMesh(core_axis_name="c", subcore_axis_name="s")
    fn = pl.kernel(
        _sc_body,
        out_type=(jax.ShapeDtypeStruct((2 * _N, _D), jnp.float32),
                  jax.ShapeDtypeStruct((2 * _N,), jnp.float32)),
        mesh=mesh,
        scratch_types=[
            pltpu.VMEM((_NIDX, 1, _G), jnp.int32),
            pltpu.VMEM((_NIDX, 1, _G), jnp.int32),
            pltpu.VMEM((_NBUF, _G, _D), jnp.float32),
            pltpu.VMEM((16, _D), jnp.float32),
            pltpu.VMEM((1280,), jnp.float32),
            pltpu.VMEM((_G,), jnp.float32),
            pltpu.VMEM_SHARED((_NA, _D), jnp.float32),
            pltpu.VMEM_SHARED((_DEGN,), jnp.float32),
            pltpu.SemaphoreType.DMA((_NIDX,)),
            pltpu.SemaphoreType.DMA((_NBUF,)),
            pltpu.SemaphoreType.DMA((_NBUF,)),
            pltpu.SemaphoreType.DMA((_NBUF,)),
        ],
    )
    return fn(feat, src3, dst3)


def _tc_combine_body(p0_ref, p1_ref, d0_ref, d1_ref, f_ref, o_ref):
    d = d0_ref[...] + d1_ref[...]
    o_ref[...] = p0_ref[...] + p1_ref[...] + d * f_ref[...]


def _tc_combine(p2, dg2, feat):
    blk = 400
    nb = _N // blk
    dg2 = dg2.reshape(2 * _N, 1)
    return pl.pallas_call(
        _tc_combine_body,
        out_shape=jax.ShapeDtypeStruct((_N, _D), jnp.float32),
        grid=(nb,),
        in_specs=[
            pl.BlockSpec((blk, _D), lambda i: (i, 0)),
            pl.BlockSpec((blk, _D), lambda i: (i + nb, 0)),
            pl.BlockSpec((blk, 1), lambda i: (i, 0)),
            pl.BlockSpec((blk, 1), lambda i: (i + nb, 0)),
            pl.BlockSpec((blk, _D), lambda i: (i, 0)),
        ],
        out_specs=pl.BlockSpec((blk, _D), lambda i: (i, 0)),
    )(p2, p2, dg2, dg2, feat)


def kernel(node_feat, edge_index):
    src3 = edge_index[0].reshape(_ER, 1, _G)
    dst3 = edge_index[1].reshape(_ER, 1, _G)
    p2, dg2 = _sc_part(node_feat, src3, dst3)
    return _tc_combine(p2, dg2, node_feat)
